# baseline (speedup 1.0000x reference)
import jax
import jax.numpy as jnp
from jax import lax
from jax.experimental import pallas as pl
from jax.experimental.pallas import tpu as pltpu

_IMG = 640.0
_MAX_DET = 128
_N = 16800
_ROWS = 132


def _topk_body(s_ref, sig_ref, idx_ref, ndet_ref, S):
    S[...] = s_ref[0]
    ids0 = lax.broadcasted_iota(jnp.int32, (_ROWS, 1), 0)
    li = lax.broadcasted_iota(jnp.int32, (1, 128), 1)
    M0 = jnp.max(S[...], axis=1, keepdims=True)

    def step(j, carry):
        M, sv, iv = carry
        gm = jnp.max(M)
        r = jnp.min(jnp.where(M == gm, ids0, jnp.int32(100000)))
        row = S[pl.ds(r, 1), :]
        c = jnp.min(jnp.where(row == gm, li, jnp.int32(100000)))
        flat = r * 128 + c
        sv = jnp.where(li == j, gm, sv)
        iv = jnp.where(li == j, flat, iv)
        row2 = jnp.where(li == c, -jnp.inf, row)
        S[pl.ds(r, 1), :] = row2
        nm = jnp.max(row2)
        M = jnp.where(ids0 == r, nm, M)
        return M, sv, iv

    _, sv, iv = lax.fori_loop(
        0, _MAX_DET, step,
        (M0, jnp.full((1, 128), -jnp.inf, jnp.float32),
         jnp.zeros((1, 128), jnp.int32)))
    sig = 1.0 / (1.0 + jnp.exp(-sv))
    sig_ref[0] = sig
    idx_ref[0] = iv
    ndet_ref[0, 0, 0] = jnp.sum((sig > 0.5).astype(jnp.int32))


def kernel(scores_8, boxes_8, landmarks_8, scores_16, boxes_16, landmarks_16,
           scores_32, boxes_32, landmarks_32, anchor_centers, anchor_strides):
    B = scores_8.shape[0]
    scores = jnp.concatenate(
        [scores_8.reshape(B, -1), scores_16.reshape(B, -1),
         scores_32.reshape(B, -1)], axis=1)
    scores = jnp.pad(scores, ((0, 0), (0, _ROWS * 128 - _N)),
                     constant_values=-jnp.inf).reshape(B, _ROWS, 128)

    sig, idx, ndet = pl.pallas_call(
        _topk_body,
        grid=(B,),
        in_specs=[pl.BlockSpec((1, _ROWS, 128), lambda b: (b, 0, 0))],
        out_specs=[
            pl.BlockSpec((1, 1, 128), lambda b: (b, 0, 0)),
            pl.BlockSpec((1, 1, 128), lambda b: (b, 0, 0)),
            pl.BlockSpec((1, 1, 1), lambda b: (b, 0, 0),
                         memory_space=pltpu.SMEM),
        ],
        out_shape=[
            jax.ShapeDtypeStruct((B, 1, 128), jnp.float32),
            jax.ShapeDtypeStruct((B, 1, 128), jnp.int32),
            jax.ShapeDtypeStruct((B, 1, 1), jnp.int32),
        ],
        scratch_shapes=[pltpu.VMEM((_ROWS, 128), jnp.float32)],
    )(scores)
    ndet = ndet.reshape(B, 1)
    sig = sig.reshape(B, 128)
    idx = idx.reshape(B, 128)

    j8 = jnp.clip(idx, 0, 12799)
    j16 = jnp.clip(idx - 12800, 0, 3199)
    j32 = jnp.clip(idx - 16000, 0, 799)
    in16 = (idx >= 12800)[..., None]
    in32 = (idx >= 16000)[..., None]

    def gather(a8, a16, a32):
        g8 = jnp.take_along_axis(a8, j8[..., None], axis=1)
        g16 = jnp.take_along_axis(a16, j16[..., None], axis=1)
        g32 = jnp.take_along_axis(a32, j32[..., None], axis=1)
        return jnp.where(in32, g32, jnp.where(in16, g16, g8))

    bx = gather(boxes_8, boxes_16, boxes_32)
    lm = gather(landmarks_8, landmarks_16, landmarks_32)
    ctr = jnp.take(anchor_centers, idx, axis=0)
    st = jnp.take(anchor_strides, idx, axis=0)[..., None]

    x1 = ctr[..., 0:1] - bx[..., 0:1] * st
    y1 = ctr[..., 1:2] - bx[..., 1:2] * st
    x2 = ctr[..., 0:1] + bx[..., 2:3] * st
    y2 = ctr[..., 1:2] + bx[..., 3:4] * st
    det_boxes = jnp.concatenate([x1, y1, x2, y2], axis=-1) / _IMG
    lmk = lm.reshape(B, 128, 5, 2)
    det_landmarks = (lmk * st[..., None] + ctr[:, :, None, :]).reshape(
        B, 128, 10) / _IMG
    return (ndet, det_boxes, sig, det_landmarks)


# speedup vs baseline: 4.1353x; 4.1353x over previous
import jax
import jax.numpy as jnp
from jax import lax
from jax.experimental import pallas as pl
from jax.experimental.pallas import tpu as pltpu

_IMG = 640.0
_MAX_DET = 128
_N = 16800
_ROWS = 132


def _topk_body(s_ref, sig_ref, idx_ref, ndet_ref, S):
    B = 16
    S[...] = s_ref[...]
    cio = lax.broadcasted_iota(jnp.int32, (B, _ROWS), 1)
    li = lax.broadcasted_iota(jnp.int32, (B, 128), 1)
    M0 = jnp.max(S[...], axis=2)

    def step(j, carry):
        M, sv, iv = carry
        gm = jnp.max(M, axis=1, keepdims=True)
        rvec = jnp.min(jnp.where(M == gm, cio, jnp.int32(100000)),
                       axis=1, keepdims=True)
        rbs = [rvec[b, 0] for b in range(B)]
        R = jnp.concatenate(
            [S[b, pl.ds(rbs[b], 1), :] for b in range(B)], axis=0)
        cvec = jnp.min(jnp.where(R == gm, li, jnp.int32(100000)),
                       axis=1, keepdims=True)
        sv = jnp.where(li == j, gm, sv)
        iv = jnp.where(li == j, rvec * 128 + cvec, iv)
        R2 = jnp.where(li == cvec, -jnp.inf, R)
        for b in range(B):
            S[b, pl.ds(rbs[b], 1), :] = R2[b:b + 1, :]
        nm = jnp.max(R2, axis=1, keepdims=True)
        M = jnp.where(cio == rvec, nm, M)
        return M, sv, iv

    _, sv, iv = lax.fori_loop(
        0, _MAX_DET, step,
        (M0, jnp.full((B, 128), -jnp.inf, jnp.float32),
         jnp.zeros((B, 128), jnp.int32)))
    sig = 1.0 / (1.0 + jnp.exp(-sv))
    sig_ref[...] = sig
    idx_ref[...] = iv
    ndet_ref[...] = jnp.sum(jnp.where(sig > 0.5, 1, 0), axis=1,
                            keepdims=True).astype(jnp.int32)


def kernel(scores_8, boxes_8, landmarks_8, scores_16, boxes_16, landmarks_16,
           scores_32, boxes_32, landmarks_32, anchor_centers, anchor_strides):
    B = scores_8.shape[0]
    scores = jnp.concatenate(
        [scores_8.reshape(B, -1), scores_16.reshape(B, -1),
         scores_32.reshape(B, -1)], axis=1)
    scores = jnp.pad(scores, ((0, 0), (0, _ROWS * 128 - _N)),
                     constant_values=-jnp.inf).reshape(B, _ROWS, 128)

    sig, idx, ndet = pl.pallas_call(
        _topk_body,
        out_shape=[
            jax.ShapeDtypeStruct((B, 128), jnp.float32),
            jax.ShapeDtypeStruct((B, 128), jnp.int32),
            jax.ShapeDtypeStruct((B, 1), jnp.int32),
        ],
        scratch_shapes=[pltpu.VMEM((B, _ROWS, 128), jnp.float32)],
    )(scores)

    j8 = jnp.clip(idx, 0, 12799)
    j16 = jnp.clip(idx - 12800, 0, 3199)
    j32 = jnp.clip(idx - 16000, 0, 799)
    in16 = (idx >= 12800)[..., None]
    in32 = (idx >= 16000)[..., None]

    def gather(a8, a16, a32):
        g8 = jnp.take_along_axis(a8, j8[..., None], axis=1)
        g16 = jnp.take_along_axis(a16, j16[..., None], axis=1)
        g32 = jnp.take_along_axis(a32, j32[..., None], axis=1)
        return jnp.where(in32, g32, jnp.where(in16, g16, g8))

    bx = gather(boxes_8, boxes_16, boxes_32)
    lm = gather(landmarks_8, landmarks_16, landmarks_32)
    ctr = jnp.take(anchor_centers, idx, axis=0)
    st = jnp.take(anchor_strides, idx, axis=0)[..., None]

    x1 = ctr[..., 0:1] - bx[..., 0:1] * st
    y1 = ctr[..., 1:2] - bx[..., 1:2] * st
    x2 = ctr[..., 0:1] + bx[..., 2:3] * st
    y2 = ctr[..., 1:2] + bx[..., 3:4] * st
    det_boxes = jnp.concatenate([x1, y1, x2, y2], axis=-1) / _IMG
    lmk = lm.reshape(B, 128, 5, 2)
    det_landmarks = (lmk * st[..., None] + ctr[:, :, None, :]).reshape(
        B, 128, 10) / _IMG
    return (ndet, det_boxes, sig, det_landmarks)
